# baseline (device time: 94491 ns/iter reference)
import os

import jax
import jax.numpy as jnp
from jax import lax
from jax.experimental import pallas as pl
from jax.experimental.pallas import tpu as pltpu

_NO_COMM = os.environ.get("KERNEL_NO_COMM") == "1"

N_DEV = 8
SQ = 256
CHUNK = SQ // N_DEV
SKV_LOCAL = 4096
NSEG = 8
SEG = SKV_LOCAL // NSEG
HQ = 8
DH = 128
D = 1024
BLK = 64
SCALE = 0.08838834764831843
NEG = -1e9


def kernel(x, Wq, K_ext, V_ext, Wo):
    def body(x_ref, wq_ref, k_ref, v_ref, wo_ref, out_ref,
             q_all, local_o, local_l, rs_o, rs_l,
             rs_send_o, rs_send_l, rs_recv_o, rs_recv_l,
             ag_send, ag_recv):
        my = lax.axis_index("i")
        g = pl.program_id(0)

        @pl.when(g == 0)
        def _():
            barrier = pltpu.get_barrier_semaphore()
            for p in range(N_DEV):
                pl.semaphore_signal(
                    barrier, inc=1,
                    device_id=(p,), device_id_type=pl.DeviceIdType.MESH,
                )
            pl.semaphore_wait(barrier, N_DEV)

            q_all[...] = (jnp.dot(
                x_ref[0].astype(jnp.bfloat16),
                wq_ref[...].astype(jnp.bfloat16),
                preferred_element_type=jnp.float32,
            ) * SCALE).astype(jnp.bfloat16)
            local_l[...] = jnp.zeros((SQ, HQ), jnp.float32)

        qb = lax.broadcasted_iota(jnp.int32, (SQ, 1), 0) // BLK
        kb = lax.broadcasted_iota(jnp.int32, (1, SEG), 1) // BLK
        kb = kb + my * (SKV_LOCAL // BLK) + g * (SEG // BLK)
        mask = (qb == kb) | (kb == 0) | ((qb + kb) % 3 == 0)

        l_cols = lax.broadcasted_iota(jnp.int32, (SQ, HQ), 1)
        l_acc = local_l[...]
        for h in range(HQ):
            q_h = q_all[:, h * DH:(h + 1) * DH]
            k_h = k_ref[0, :, h, :].astype(jnp.bfloat16)
            s = lax.dot_general(
                q_h, k_h, (((1,), (1,)), ((), ())),
                preferred_element_type=jnp.float32,
            )
            s = jnp.where(mask, s, NEG)
            w = jnp.exp(s.astype(jnp.bfloat16))
            l_h = jnp.sum(w.astype(jnp.float32), axis=1)
            l_acc = jnp.where(l_cols == h, l_acc + l_h[:, None], l_acc)
            v_h = v_ref[0, :, h, :].astype(jnp.bfloat16)
            o_h = lax.dot_general(
                w, v_h, (((1,), (0,)), ((), ())),
                preferred_element_type=jnp.float32,
            )

            @pl.when(g == 0)
            def _():
                local_o[h] = o_h

            @pl.when(g != 0)
            def _():
                local_o[h] = local_o[h] + o_h
        local_l[...] = l_acc

        @pl.when(g == NSEG - 1)
        def _():
            for p in range(N_DEV) if not _NO_COMM else []:
                @pl.when(my != p)
                def _():
                    d_o = pltpu.make_async_remote_copy(
                        src_ref=local_o.at[:, p * CHUNK:(p + 1) * CHUNK, :],
                        dst_ref=rs_o.at[my],
                        send_sem=rs_send_o.at[p], recv_sem=rs_recv_o.at[my],
                        device_id=(p,), device_id_type=pl.DeviceIdType.MESH,
                    )
                    d_l = pltpu.make_async_remote_copy(
                        src_ref=local_l.at[p * CHUNK:(p + 1) * CHUNK, :],
                        dst_ref=rs_l.at[my],
                        send_sem=rs_send_l.at[p], recv_sem=rs_recv_l.at[my],
                        device_id=(p,), device_id_type=pl.DeviceIdType.MESH,
                    )
                    d_o.start()
                    d_l.start()

            rs_o[my] = local_o[:, pl.ds(my * CHUNK, CHUNK), :]
            rs_l[my] = local_l[pl.ds(my * CHUNK, CHUNK), :]

            for p in range(N_DEV) if not _NO_COMM else []:
                @pl.when(my != p)
                def _():
                    r_o = pltpu.make_async_remote_copy(
                        src_ref=rs_o.at[p], dst_ref=rs_o.at[p],
                        send_sem=rs_send_o.at[p], recv_sem=rs_recv_o.at[p],
                        device_id=(p,), device_id_type=pl.DeviceIdType.MESH,
                    )
                    r_l = pltpu.make_async_remote_copy(
                        src_ref=rs_l.at[p], dst_ref=rs_l.at[p],
                        send_sem=rs_send_l.at[p], recv_sem=rs_recv_l.at[p],
                        device_id=(p,), device_id_type=pl.DeviceIdType.MESH,
                    )
                    r_o.wait_recv()
                    r_l.wait_recv()

            l_sum = jnp.sum(rs_l[...], axis=0)
            l_t = jnp.transpose(l_sum, (1, 0))
            o_acc = rs_o[0]
            for p in range(1, N_DEV):
                o_acc = o_acc + rs_o[p]
            ctx = o_acc / l_t[:, :, None]

            ctx2d = jnp.concatenate([ctx[i] for i in range(HQ)], axis=1)
            out_chunk = jnp.dot(
                ctx2d.astype(jnp.bfloat16),
                wo_ref[...].astype(jnp.bfloat16),
                preferred_element_type=jnp.float32,
            )
            out_ref[0, pl.ds(my * CHUNK, CHUNK), :] = out_chunk

            for p in range(N_DEV) if not _NO_COMM else []:
                @pl.when(my != p)
                def _():
                    d_g = pltpu.make_async_remote_copy(
                        src_ref=out_ref.at[0, pl.ds(my * CHUNK, CHUNK), :],
                        dst_ref=out_ref.at[0, pl.ds(my * CHUNK, CHUNK), :],
                        send_sem=ag_send.at[p], recv_sem=ag_recv.at[my],
                        device_id=(p,), device_id_type=pl.DeviceIdType.MESH,
                    )
                    d_g.start()
            for p in range(N_DEV) if not _NO_COMM else []:
                @pl.when(my != p)
                def _():
                    r_g = pltpu.make_async_remote_copy(
                        src_ref=out_ref.at[0, pl.ds(p * CHUNK, CHUNK), :],
                        dst_ref=out_ref.at[0, pl.ds(p * CHUNK, CHUNK), :],
                        send_sem=ag_send.at[p], recv_sem=ag_recv.at[p],
                        device_id=(p,), device_id_type=pl.DeviceIdType.MESH,
                    )
                    r_g.wait_recv()

            for p in range(N_DEV) if not _NO_COMM else []:
                @pl.when(my != p)
                def _():
                    w_o = pltpu.make_async_remote_copy(
                        src_ref=local_o.at[:, p * CHUNK:(p + 1) * CHUNK, :],
                        dst_ref=rs_o.at[my],
                        send_sem=rs_send_o.at[p], recv_sem=rs_recv_o.at[my],
                        device_id=(p,), device_id_type=pl.DeviceIdType.MESH,
                    )
                    w_l = pltpu.make_async_remote_copy(
                        src_ref=local_l.at[p * CHUNK:(p + 1) * CHUNK, :],
                        dst_ref=rs_l.at[my],
                        send_sem=rs_send_l.at[p], recv_sem=rs_recv_l.at[my],
                        device_id=(p,), device_id_type=pl.DeviceIdType.MESH,
                    )
                    w_g = pltpu.make_async_remote_copy(
                        src_ref=out_ref.at[0, pl.ds(my * CHUNK, CHUNK), :],
                        dst_ref=out_ref.at[0, pl.ds(my * CHUNK, CHUNK), :],
                        send_sem=ag_send.at[p], recv_sem=ag_recv.at[my],
                        device_id=(p,), device_id_type=pl.DeviceIdType.MESH,
                    )
                    w_o.wait_send()
                    w_l.wait_send()
                    w_g.wait_send()

    return pl.pallas_call(
        body,
        grid=(NSEG,),
        out_shape=jax.ShapeDtypeStruct((1, SQ, D), jnp.float32),
        in_specs=[
            pl.BlockSpec((1, SQ, D), lambda g: (0, 0, 0)),
            pl.BlockSpec((D, D), lambda g: (0, 0)),
            pl.BlockSpec((1, SEG, HQ, DH), lambda g: (0, g, 0, 0)),
            pl.BlockSpec((1, SEG, HQ, DH), lambda g: (0, g, 0, 0)),
            pl.BlockSpec((D, D), lambda g: (0, 0)),
        ],
        out_specs=pl.BlockSpec((1, SQ, D), lambda g: (0, 0, 0)),
        scratch_shapes=[
            pltpu.VMEM((SQ, D), jnp.bfloat16),
            pltpu.VMEM((HQ, SQ, DH), jnp.float32),
            pltpu.VMEM((SQ, HQ), jnp.float32),
            pltpu.VMEM((N_DEV, HQ, CHUNK, DH), jnp.float32),
            pltpu.VMEM((N_DEV, CHUNK, HQ), jnp.float32),
            pltpu.SemaphoreType.DMA((N_DEV,)),
            pltpu.SemaphoreType.DMA((N_DEV,)),
            pltpu.SemaphoreType.DMA((N_DEV,)),
            pltpu.SemaphoreType.DMA((N_DEV,)),
            pltpu.SemaphoreType.DMA((N_DEV,)),
            pltpu.SemaphoreType.DMA((N_DEV,)),
        ],
        compiler_params=pltpu.CompilerParams(
            collective_id=0, vmem_limit_bytes=64 * 1024 * 1024,
        ),
    )(x, Wq, K_ext, V_ext, Wo)
